# baseline (speedup 1.0000x reference)
import jax
import jax.numpy as jnp
from jax import lax
from jax.experimental import pallas as pl
from jax.experimental.pallas import tpu as pltpu
from jax.experimental.pallas import tpu_sc as plsc

N_NODES = 10000
N_EDGES = 320000
D = 128
DH = 64

NC = 2
NS = 16
CH = 128
E_TILE = N_EDGES // NS
NCHUNK = E_TILE // CH
NQUAD = NCHUNK // 4
CT = E_TILE - NCHUNK * CH
N_PAD = 10240
ROWS_TILE = N_PAD // NS
ROWS_LAST = N_NODES - (NS - 1) * ROWS_TILE


def _body(x_hbm, adj_hbm, vals_hbm, z_hbm, out_hbm,
          src_v, dst_v, vals_v, rows_v, src_t, dst_t, vals_t, rows_t,
          xs, acc, sem_g, sem_s, sem_i, sem_t):
    c = lax.axis_index("c")
    s = lax.axis_index("s")

    @pl.when(s < NS - 1)
    def _():
        pltpu.sync_copy(x_hbm.at[pl.ds(s * ROWS_TILE, ROWS_TILE),
                                 pl.ds(c * DH, DH)],
                        xs.at[pl.ds(s * ROWS_TILE, ROWS_TILE)])

    @pl.when(s == NS - 1)
    def _():
        pltpu.sync_copy(x_hbm.at[pl.ds((NS - 1) * ROWS_TILE, ROWS_LAST),
                                 pl.ds(c * DH, DH)],
                        xs.at[pl.ds((NS - 1) * ROWS_TILE, ROWS_LAST)])
    pltpu.sync_copy(z_hbm.at[pl.ds(s * ROWS_TILE, ROWS_TILE)],
                    acc.at[pl.ds(s * ROWS_TILE, ROWS_TILE)])
    plsc.subcore_barrier()

    def issue_idx(i, m):
        off = s * E_TILE + i * CH
        pltpu.async_copy(adj_hbm.at[1, pl.ds(off, CH)], src_v.at[m],
                         sem_i.at[m])
        pltpu.async_copy(adj_hbm.at[0, pl.ds(off, CH)], dst_v.at[m],
                         sem_i.at[m])
        pltpu.async_copy(vals_hbm.at[pl.ds(off, CH)], vals_v.at[m],
                         sem_i.at[m])

    def wait_idx(i, m):
        off = s * E_TILE + i * CH
        pltpu.make_async_copy(adj_hbm.at[1, pl.ds(off, CH)], src_v.at[m],
                              sem_i.at[m]).wait()
        pltpu.make_async_copy(adj_hbm.at[0, pl.ds(off, CH)], dst_v.at[m],
                              sem_i.at[m]).wait()
        pltpu.make_async_copy(vals_hbm.at[pl.ds(off, CH)], vals_v.at[m],
                              sem_i.at[m]).wait()

    def issue_gather(m):
        pltpu.async_copy(xs.at[src_v.at[m]], rows_v.at[m], sem_g.at[m])

    def wait_gather(m):
        pltpu.make_async_copy(xs.at[src_v.at[m]], rows_v.at[m],
                              sem_g.at[m]).wait()

    def issue_scatter(m):
        pltpu.async_copy(rows_v.at[m], acc.at[dst_v.at[m]], sem_s.at[m],
                         add=True)

    def wait_scatter(m):
        pltpu.make_async_copy(rows_v.at[m], acc.at[dst_v.at[m]],
                              sem_s.at[m]).wait()

    def scale(m):
        def grp(g, carry):
            vgrp = vals_v[m, pl.ds(g * 16, 16)]
            for u in range(16):
                e = g * 16 + u
                vv = vgrp[u]
                for f in range(DH // 16):
                    sl = pl.ds(f * 16, 16)
                    rows_v[m, e, sl] = rows_v[m, e, sl] * vv
            return carry

        lax.fori_loop(0, CH // 16, grp, 0, unroll=True)

    issue_idx(0, 0)
    issue_idx(1, 1)
    wait_idx(0, 0)
    issue_gather(0)

    def quad(t, carry):
        i4 = t * 4
        for k in range(4):
            i = i4 + k

            def drain_prev():
                wait_scatter((k + 2) % 4)

            if k < 2:
                @pl.when(t > 0)
                def _():
                    drain_prev()
            else:
                drain_prev()

            if k < 2:
                issue_idx(i + 2, (k + 2) % 4)
            else:
                @pl.when(t < NQUAD - 1)
                def _():
                    issue_idx(i + 2, (k + 2) % 4)

            def start_next():
                wait_idx(i + 1, (k + 1) % 4)
                issue_gather((k + 1) % 4)

            if k < 3:
                start_next()
            else:
                @pl.when(t < NQUAD - 1)
                def _():
                    start_next()

            wait_gather(k)
            scale(k)
            issue_scatter(k)
        return carry

    lax.fori_loop(0, NQUAD, quad, 0, unroll=False)

    toff = s * E_TILE + NCHUNK * CH
    pltpu.async_copy(adj_hbm.at[1, pl.ds(toff, CT)], src_t, sem_t)
    pltpu.async_copy(adj_hbm.at[0, pl.ds(toff, CT)], dst_t, sem_t)
    pltpu.async_copy(vals_hbm.at[pl.ds(toff, CT)], vals_t, sem_t)
    pltpu.make_async_copy(adj_hbm.at[1, pl.ds(toff, CT)], src_t, sem_t).wait()
    pltpu.make_async_copy(adj_hbm.at[0, pl.ds(toff, CT)], dst_t, sem_t).wait()
    pltpu.make_async_copy(vals_hbm.at[pl.ds(toff, CT)], vals_t, sem_t).wait()
    pltpu.async_copy(xs.at[src_t], rows_t, sem_t)
    pltpu.make_async_copy(xs.at[src_t], rows_t, sem_t).wait()
    for g in range(CT // 16):
        vgrp_t = vals_t[pl.ds(g * 16, 16)]
        for u in range(16):
            e = g * 16 + u
            vv = vgrp_t[u]
            for f in range(DH // 16):
                sl = pl.ds(f * 16, 16)
                rows_t[e, sl] = rows_t[e, sl] * vv
    pltpu.async_copy(rows_t, acc.at[dst_t], sem_t, add=True)

    wait_scatter(2)
    wait_scatter(3)
    pltpu.make_async_copy(rows_t, acc.at[dst_t], sem_t).wait()

    plsc.subcore_barrier()

    @pl.when(s < NS - 1)
    def _():
        pltpu.sync_copy(acc.at[pl.ds(s * ROWS_TILE, ROWS_TILE)],
                        out_hbm.at[pl.ds(s * ROWS_TILE, ROWS_TILE),
                                   pl.ds(c * DH, DH)])

    @pl.when(s == NS - 1)
    def _():
        pltpu.sync_copy(acc.at[pl.ds((NS - 1) * ROWS_TILE, ROWS_LAST)],
                        out_hbm.at[pl.ds((NS - 1) * ROWS_TILE, ROWS_LAST),
                                   pl.ds(c * DH, DH)])


@jax.jit
def _spmm(x, adj, vals):
    z = jnp.zeros((N_PAD, DH), jnp.float32)

    mesh = plsc.VectorSubcoreMesh(core_axis_name="c", subcore_axis_name="s")
    out = pl.kernel(
        _body,
        out_type=jax.ShapeDtypeStruct((N_NODES, D), jnp.float32),
        mesh=mesh,
        compiler_params=pltpu.CompilerParams(use_tc_tiling_on_sc=False),
        scratch_types=[
            pltpu.VMEM((4, CH), jnp.int32),
            pltpu.VMEM((4, CH), jnp.int32),
            pltpu.VMEM((4, CH), jnp.float32),
            pltpu.VMEM((4, CH, DH), jnp.float32),
            pltpu.VMEM((CT,), jnp.int32),
            pltpu.VMEM((CT,), jnp.int32),
            pltpu.VMEM((CT,), jnp.float32),
            pltpu.VMEM((CT, DH), jnp.float32),
            pltpu.VMEM_SHARED((N_PAD, DH), jnp.float32),
            pltpu.VMEM_SHARED((N_PAD, DH), jnp.float32),
            pltpu.SemaphoreType.DMA((4,)),
            pltpu.SemaphoreType.DMA((4,)),
            pltpu.SemaphoreType.DMA((4,)),
            pltpu.SemaphoreType.DMA,
        ],
    )(x, adj, vals, z)
    return out


def kernel(x, adj_indices, adj_values, idx):
    del idx
    return _spmm(x, adj_indices.astype(jnp.int32), adj_values)


# speedup vs baseline: 1.1845x; 1.1845x over previous
import jax
import jax.numpy as jnp
from jax import lax
from jax.experimental import pallas as pl
from jax.experimental.pallas import tpu as pltpu
from jax.experimental.pallas import tpu_sc as plsc

N_NODES = 10000
N_EDGES = 320000
D = 128
DH = 64

NC = 2
NS = 16
CH = 128
E_TILE = N_EDGES // NS
NCHUNK = E_TILE // CH
NQUAD = NCHUNK // 4
CT = E_TILE - NCHUNK * CH
N_PAD = 10240
ROWS_TILE = N_PAD // NS
ROWS_LAST = N_NODES - (NS - 1) * ROWS_TILE


def _body(x_hbm, adj_hbm, vals_hbm, z_hbm, out_hbm,
          src_v, dst_v, vals_v, rows_v, src_t, dst_t, vals_t, rows_t,
          xs, acc, sem_g, sem_s, sem_i, sem_t):
    c = lax.axis_index("c")
    s = lax.axis_index("s")

    @pl.when(s < NS - 1)
    def _():
        pltpu.sync_copy(x_hbm.at[pl.ds(s * ROWS_TILE, ROWS_TILE),
                                 pl.ds(c * DH, DH)],
                        xs.at[pl.ds(s * ROWS_TILE, ROWS_TILE)])

    @pl.when(s == NS - 1)
    def _():
        pltpu.sync_copy(x_hbm.at[pl.ds((NS - 1) * ROWS_TILE, ROWS_LAST),
                                 pl.ds(c * DH, DH)],
                        xs.at[pl.ds((NS - 1) * ROWS_TILE, ROWS_LAST)])
    pltpu.sync_copy(z_hbm.at[pl.ds(s * ROWS_TILE, ROWS_TILE)],
                    acc.at[pl.ds(s * ROWS_TILE, ROWS_TILE)])
    plsc.subcore_barrier()

    def issue_idx(i, m):
        off = s * E_TILE + i * CH
        pltpu.async_copy(adj_hbm.at[1, pl.ds(off, CH)], src_v.at[m],
                         sem_i.at[m])
        pltpu.async_copy(adj_hbm.at[0, pl.ds(off, CH)], dst_v.at[m],
                         sem_i.at[m])
        pltpu.async_copy(vals_hbm.at[pl.ds(off, CH)], vals_v.at[m],
                         sem_i.at[m])

    def wait_idx(i, m):
        off = s * E_TILE + i * CH
        pltpu.make_async_copy(adj_hbm.at[1, pl.ds(off, CH)], src_v.at[m],
                              sem_i.at[m]).wait()
        pltpu.make_async_copy(adj_hbm.at[0, pl.ds(off, CH)], dst_v.at[m],
                              sem_i.at[m]).wait()
        pltpu.make_async_copy(vals_hbm.at[pl.ds(off, CH)], vals_v.at[m],
                              sem_i.at[m]).wait()

    def issue_gather(m):
        pltpu.async_copy(xs.at[src_v.at[m]], rows_v.at[m], sem_g.at[m])

    def wait_gather(m):
        pltpu.make_async_copy(xs.at[src_v.at[m]], rows_v.at[m],
                              sem_g.at[m]).wait()

    def issue_scatter(m):
        pltpu.async_copy(rows_v.at[m], acc.at[dst_v.at[m]], sem_s.at[m],
                         add=True)

    def wait_scatter(m):
        pltpu.make_async_copy(rows_v.at[m], acc.at[dst_v.at[m]],
                              sem_s.at[m]).wait()

    def scale(m):
        def grp(g, carry):
            vgrp = vals_v[m, pl.ds(g * 16, 16)]
            for u in range(16):
                e = g * 16 + u
                vv = vgrp[u]
                for f in range(DH // 16):
                    sl = pl.ds(f * 16, 16)
                    rows_v[m, e, sl] = rows_v[m, e, sl] * vv
            return carry

        lax.fori_loop(0, CH // 16, grp, 0, unroll=4)

    issue_idx(0, 0)
    issue_idx(1, 1)
    wait_idx(0, 0)
    issue_gather(0)

    def quad(t, carry):
        i4 = t * 4
        for k in range(4):
            i = i4 + k

            def drain_prev():
                wait_scatter((k + 2) % 4)

            if k < 2:
                @pl.when(t > 0)
                def _():
                    drain_prev()
            else:
                drain_prev()

            if k < 2:
                issue_idx(i + 2, (k + 2) % 4)
            else:
                @pl.when(t < NQUAD - 1)
                def _():
                    issue_idx(i + 2, (k + 2) % 4)

            def start_next():
                wait_idx(i + 1, (k + 1) % 4)
                issue_gather((k + 1) % 4)

            if k < 3:
                start_next()
            else:
                @pl.when(t < NQUAD - 1)
                def _():
                    start_next()

            wait_gather(k)
            scale(k)
            issue_scatter(k)
        return carry

    lax.fori_loop(0, NQUAD, quad, 0, unroll=False)

    toff = s * E_TILE + NCHUNK * CH
    pltpu.async_copy(adj_hbm.at[1, pl.ds(toff, CT)], src_t, sem_t)
    pltpu.async_copy(adj_hbm.at[0, pl.ds(toff, CT)], dst_t, sem_t)
    pltpu.async_copy(vals_hbm.at[pl.ds(toff, CT)], vals_t, sem_t)
    pltpu.make_async_copy(adj_hbm.at[1, pl.ds(toff, CT)], src_t, sem_t).wait()
    pltpu.make_async_copy(adj_hbm.at[0, pl.ds(toff, CT)], dst_t, sem_t).wait()
    pltpu.make_async_copy(vals_hbm.at[pl.ds(toff, CT)], vals_t, sem_t).wait()
    pltpu.async_copy(xs.at[src_t], rows_t, sem_t)
    pltpu.make_async_copy(xs.at[src_t], rows_t, sem_t).wait()
    for g in range(CT // 16):
        vgrp_t = vals_t[pl.ds(g * 16, 16)]
        for u in range(16):
            e = g * 16 + u
            vv = vgrp_t[u]
            for f in range(DH // 16):
                sl = pl.ds(f * 16, 16)
                rows_t[e, sl] = rows_t[e, sl] * vv
    pltpu.async_copy(rows_t, acc.at[dst_t], sem_t, add=True)

    wait_scatter(2)
    wait_scatter(3)
    pltpu.make_async_copy(rows_t, acc.at[dst_t], sem_t).wait()

    plsc.subcore_barrier()

    @pl.when(s < NS - 1)
    def _():
        pltpu.sync_copy(acc.at[pl.ds(s * ROWS_TILE, ROWS_TILE)],
                        out_hbm.at[pl.ds(s * ROWS_TILE, ROWS_TILE),
                                   pl.ds(c * DH, DH)])

    @pl.when(s == NS - 1)
    def _():
        pltpu.sync_copy(acc.at[pl.ds((NS - 1) * ROWS_TILE, ROWS_LAST)],
                        out_hbm.at[pl.ds((NS - 1) * ROWS_TILE, ROWS_LAST),
                                   pl.ds(c * DH, DH)])


@jax.jit
def _spmm(x, adj, vals):
    z = jnp.zeros((N_PAD, DH), jnp.float32)

    mesh = plsc.VectorSubcoreMesh(core_axis_name="c", subcore_axis_name="s")
    out = pl.kernel(
        _body,
        out_type=jax.ShapeDtypeStruct((N_NODES, D), jnp.float32),
        mesh=mesh,
        compiler_params=pltpu.CompilerParams(use_tc_tiling_on_sc=False),
        scratch_types=[
            pltpu.VMEM((4, CH), jnp.int32),
            pltpu.VMEM((4, CH), jnp.int32),
            pltpu.VMEM((4, CH), jnp.float32),
            pltpu.VMEM((4, CH, DH), jnp.float32),
            pltpu.VMEM((CT,), jnp.int32),
            pltpu.VMEM((CT,), jnp.int32),
            pltpu.VMEM((CT,), jnp.float32),
            pltpu.VMEM((CT, DH), jnp.float32),
            pltpu.VMEM_SHARED((N_PAD, DH), jnp.float32),
            pltpu.VMEM_SHARED((N_PAD, DH), jnp.float32),
            pltpu.SemaphoreType.DMA((4,)),
            pltpu.SemaphoreType.DMA((4,)),
            pltpu.SemaphoreType.DMA((4,)),
            pltpu.SemaphoreType.DMA,
        ],
    )(x, adj, vals, z)
    return out


def kernel(x, adj_indices, adj_values, idx):
    del idx
    return _spmm(x, adj_indices.astype(jnp.int32), adj_values)


# speedup vs baseline: 1.1906x; 1.0052x over previous
import jax
import jax.numpy as jnp
from jax import lax
from jax.experimental import pallas as pl
from jax.experimental.pallas import tpu as pltpu
from jax.experimental.pallas import tpu_sc as plsc

N_NODES = 10000
N_EDGES = 320000
D = 128
DH = 64

NC = 2
NS = 16
CH = 128
E_TILE = N_EDGES // NS
NCHUNK = E_TILE // CH
NQUAD = NCHUNK // 4
CT = E_TILE - NCHUNK * CH
N_PAD = 10240
ROWS_TILE = N_PAD // NS
ROWS_LAST = N_NODES - (NS - 1) * ROWS_TILE


def _body(x_hbm, adj_hbm, vals_hbm, z_hbm, out_hbm,
          sd_v, vals_v, rows_v, src_t, dst_t, vals_t, rows_t,
          xs, acc, sem_g, sem_s, sem_i, sem_t):
    c = lax.axis_index("c")
    s = lax.axis_index("s")

    @pl.when(s < NS - 1)
    def _():
        pltpu.sync_copy(x_hbm.at[pl.ds(s * ROWS_TILE, ROWS_TILE),
                                 pl.ds(c * DH, DH)],
                        xs.at[pl.ds(s * ROWS_TILE, ROWS_TILE)])

    @pl.when(s == NS - 1)
    def _():
        pltpu.sync_copy(x_hbm.at[pl.ds((NS - 1) * ROWS_TILE, ROWS_LAST),
                                 pl.ds(c * DH, DH)],
                        xs.at[pl.ds((NS - 1) * ROWS_TILE, ROWS_LAST)])
    pltpu.sync_copy(z_hbm.at[pl.ds(s * ROWS_TILE, ROWS_TILE)],
                    acc.at[pl.ds(s * ROWS_TILE, ROWS_TILE)])
    plsc.subcore_barrier()

    def issue_idx(i, m):
        off = s * E_TILE + i * CH
        pltpu.async_copy(adj_hbm.at[:, pl.ds(off, CH)], sd_v.at[m],
                         sem_i.at[m])
        pltpu.async_copy(vals_hbm.at[pl.ds(off, CH)], vals_v.at[m],
                         sem_i.at[m])

    def wait_idx(i, m):
        off = s * E_TILE + i * CH
        pltpu.make_async_copy(adj_hbm.at[:, pl.ds(off, CH)], sd_v.at[m],
                              sem_i.at[m]).wait()
        pltpu.make_async_copy(vals_hbm.at[pl.ds(off, CH)], vals_v.at[m],
                              sem_i.at[m]).wait()

    def issue_gather(m):
        pltpu.async_copy(xs.at[sd_v.at[m, 1]], rows_v.at[m], sem_g.at[m])

    def wait_gather(m):
        pltpu.make_async_copy(xs.at[sd_v.at[m, 1]], rows_v.at[m],
                              sem_g.at[m]).wait()

    def issue_scatter(m):
        pltpu.async_copy(rows_v.at[m], acc.at[sd_v.at[m, 0]], sem_s.at[m],
                         add=True)

    def wait_scatter(m):
        pltpu.make_async_copy(rows_v.at[m], acc.at[sd_v.at[m, 0]],
                              sem_s.at[m]).wait()

    def scale(m):
        def grp(g, carry):
            vgrp = vals_v[m, pl.ds(g * 16, 16)]
            for u in range(16):
                e = g * 16 + u
                vv = vgrp[u]
                for f in range(DH // 16):
                    sl = pl.ds(f * 16, 16)
                    rows_v[m, e, sl] = rows_v[m, e, sl] * vv
            return carry

        lax.fori_loop(0, CH // 16, grp, 0, unroll=2)

    issue_idx(0, 0)
    issue_idx(1, 1)
    wait_idx(0, 0)
    issue_gather(0)

    def quad(t, carry):
        i4 = t * 4
        for k in range(4):
            i = i4 + k

            def drain_prev():
                wait_scatter((k + 2) % 4)

            if k < 2:
                @pl.when(t > 0)
                def _():
                    drain_prev()
            else:
                drain_prev()

            if k < 2:
                issue_idx(i + 2, (k + 2) % 4)
            else:
                @pl.when(t < NQUAD - 1)
                def _():
                    issue_idx(i + 2, (k + 2) % 4)

            def start_next():
                wait_idx(i + 1, (k + 1) % 4)
                issue_gather((k + 1) % 4)

            if k < 3:
                start_next()
            else:
                @pl.when(t < NQUAD - 1)
                def _():
                    start_next()

            wait_gather(k)
            scale(k)
            issue_scatter(k)
        return carry

    lax.fori_loop(0, NQUAD, quad, 0, unroll=False)

    toff = s * E_TILE + NCHUNK * CH
    pltpu.async_copy(adj_hbm.at[1, pl.ds(toff, CT)], src_t, sem_t)
    pltpu.async_copy(adj_hbm.at[0, pl.ds(toff, CT)], dst_t, sem_t)
    pltpu.async_copy(vals_hbm.at[pl.ds(toff, CT)], vals_t, sem_t)
    pltpu.make_async_copy(adj_hbm.at[1, pl.ds(toff, CT)], src_t, sem_t).wait()
    pltpu.make_async_copy(adj_hbm.at[0, pl.ds(toff, CT)], dst_t, sem_t).wait()
    pltpu.make_async_copy(vals_hbm.at[pl.ds(toff, CT)], vals_t, sem_t).wait()
    pltpu.async_copy(xs.at[src_t], rows_t, sem_t)
    pltpu.make_async_copy(xs.at[src_t], rows_t, sem_t).wait()
    for g in range(CT // 16):
        vgrp_t = vals_t[pl.ds(g * 16, 16)]
        for u in range(16):
            e = g * 16 + u
            vv = vgrp_t[u]
            for f in range(DH // 16):
                sl = pl.ds(f * 16, 16)
                rows_t[e, sl] = rows_t[e, sl] * vv
    pltpu.async_copy(rows_t, acc.at[dst_t], sem_t, add=True)

    wait_scatter(2)
    wait_scatter(3)
    pltpu.make_async_copy(rows_t, acc.at[dst_t], sem_t).wait()

    plsc.subcore_barrier()

    @pl.when(s < NS - 1)
    def _():
        pltpu.sync_copy(acc.at[pl.ds(s * ROWS_TILE, ROWS_TILE)],
                        out_hbm.at[pl.ds(s * ROWS_TILE, ROWS_TILE),
                                   pl.ds(c * DH, DH)])

    @pl.when(s == NS - 1)
    def _():
        pltpu.sync_copy(acc.at[pl.ds((NS - 1) * ROWS_TILE, ROWS_LAST)],
                        out_hbm.at[pl.ds((NS - 1) * ROWS_TILE, ROWS_LAST),
                                   pl.ds(c * DH, DH)])


@jax.jit
def _spmm(x, adj, vals):
    z = jnp.zeros((N_PAD, DH), jnp.float32)

    mesh = plsc.VectorSubcoreMesh(core_axis_name="c", subcore_axis_name="s")
    out = pl.kernel(
        _body,
        out_type=jax.ShapeDtypeStruct((N_NODES, D), jnp.float32),
        mesh=mesh,
        compiler_params=pltpu.CompilerParams(use_tc_tiling_on_sc=False),
        scratch_types=[
            pltpu.VMEM((4, 2, CH), jnp.int32),
            pltpu.VMEM((4, CH), jnp.float32),
            pltpu.VMEM((4, CH, DH), jnp.float32),
            pltpu.VMEM((CT,), jnp.int32),
            pltpu.VMEM((CT,), jnp.int32),
            pltpu.VMEM((CT,), jnp.float32),
            pltpu.VMEM((CT, DH), jnp.float32),
            pltpu.VMEM_SHARED((N_PAD, DH), jnp.float32),
            pltpu.VMEM_SHARED((N_PAD, DH), jnp.float32),
            pltpu.SemaphoreType.DMA((4,)),
            pltpu.SemaphoreType.DMA((4,)),
            pltpu.SemaphoreType.DMA((4,)),
            pltpu.SemaphoreType.DMA,
        ],
    )(x, adj, vals, z)
    return out


def kernel(x, adj_indices, adj_values, idx):
    del idx
    return _spmm(x, adj_indices.astype(jnp.int32), adj_values)


# speedup vs baseline: 1.1920x; 1.0012x over previous
import jax
import jax.numpy as jnp
from jax import lax
from jax.experimental import pallas as pl
from jax.experimental.pallas import tpu as pltpu
from jax.experimental.pallas import tpu_sc as plsc

N_NODES = 10000
N_EDGES = 320000
D = 128
DH = 64

NC = 2
NS = 16
CH = 128
E_TILE = N_EDGES // NS
NCHUNK = E_TILE // CH
NQUAD = NCHUNK // 4
CT = E_TILE - NCHUNK * CH
N_PAD = 10240
ROWS_TILE = N_PAD // NS
ROWS_LAST = N_NODES - (NS - 1) * ROWS_TILE


def _body(x_hbm, adj_hbm, vals_hbm, z_hbm, out_hbm,
          sd_v, vals_v, rows_v, src_t, dst_t, vals_t, rows_t,
          xs, acc, sem_g, sem_s, sem_i, sem_t):
    c = lax.axis_index("c")
    s = lax.axis_index("s")

    @pl.when(s < NS - 1)
    def _():
        pltpu.sync_copy(x_hbm.at[pl.ds(s * ROWS_TILE, ROWS_TILE),
                                 pl.ds(c * DH, DH)],
                        xs.at[pl.ds(s * ROWS_TILE, ROWS_TILE)])

    @pl.when(s == NS - 1)
    def _():
        pltpu.sync_copy(x_hbm.at[pl.ds((NS - 1) * ROWS_TILE, ROWS_LAST),
                                 pl.ds(c * DH, DH)],
                        xs.at[pl.ds((NS - 1) * ROWS_TILE, ROWS_LAST)])
    pltpu.sync_copy(z_hbm, acc.at[pl.ds(s * ROWS_TILE, ROWS_TILE)])
    plsc.subcore_barrier()

    def issue_idx(i, m):
        off = s * E_TILE + i * CH
        pltpu.async_copy(adj_hbm.at[:, pl.ds(off, CH)], sd_v.at[m],
                         sem_i.at[m])
        pltpu.async_copy(vals_hbm.at[pl.ds(off, CH)], vals_v.at[m],
                         sem_i.at[m])

    def wait_idx(i, m):
        off = s * E_TILE + i * CH
        pltpu.make_async_copy(adj_hbm.at[:, pl.ds(off, CH)], sd_v.at[m],
                              sem_i.at[m]).wait()
        pltpu.make_async_copy(vals_hbm.at[pl.ds(off, CH)], vals_v.at[m],
                              sem_i.at[m]).wait()

    def issue_gather(m):
        pltpu.async_copy(xs.at[sd_v.at[m, 1]], rows_v.at[m], sem_g.at[m])

    def wait_gather(m):
        pltpu.make_async_copy(xs.at[sd_v.at[m, 1]], rows_v.at[m],
                              sem_g.at[m]).wait()

    def issue_scatter(m):
        pltpu.async_copy(rows_v.at[m], acc.at[sd_v.at[m, 0]], sem_s.at[m],
                         add=True)

    def wait_scatter(m):
        pltpu.make_async_copy(rows_v.at[m], acc.at[sd_v.at[m, 0]],
                              sem_s.at[m]).wait()

    def scale(m):
        def grp(g, carry):
            vgrp = vals_v[m, pl.ds(g * 16, 16)]
            for u in range(16):
                e = g * 16 + u
                vv = vgrp[u]
                for f in range(DH // 16):
                    sl = pl.ds(f * 16, 16)
                    rows_v[m, e, sl] = rows_v[m, e, sl] * vv
            return carry

        lax.fori_loop(0, CH // 16, grp, 0, unroll=2)

    issue_idx(0, 0)
    issue_idx(1, 1)
    wait_idx(0, 0)
    issue_gather(0)

    def quad(t, carry):
        i4 = t * 4
        for k in range(4):
            i = i4 + k

            def drain_prev():
                wait_scatter((k + 2) % 4)

            if k < 2:
                @pl.when(t > 0)
                def _():
                    drain_prev()
            else:
                drain_prev()

            if k < 2:
                issue_idx(i + 2, (k + 2) % 4)
            else:
                @pl.when(t < NQUAD - 1)
                def _():
                    issue_idx(i + 2, (k + 2) % 4)

            def start_next():
                wait_idx(i + 1, (k + 1) % 4)
                issue_gather((k + 1) % 4)

            if k < 3:
                start_next()
            else:
                @pl.when(t < NQUAD - 1)
                def _():
                    start_next()

            wait_gather(k)
            scale(k)
            issue_scatter(k)
        return carry

    lax.fori_loop(0, NQUAD, quad, 0, unroll=False)

    toff = s * E_TILE + NCHUNK * CH
    pltpu.async_copy(adj_hbm.at[1, pl.ds(toff, CT)], src_t, sem_t)
    pltpu.async_copy(adj_hbm.at[0, pl.ds(toff, CT)], dst_t, sem_t)
    pltpu.async_copy(vals_hbm.at[pl.ds(toff, CT)], vals_t, sem_t)
    pltpu.make_async_copy(adj_hbm.at[1, pl.ds(toff, CT)], src_t, sem_t).wait()
    pltpu.make_async_copy(adj_hbm.at[0, pl.ds(toff, CT)], dst_t, sem_t).wait()
    pltpu.make_async_copy(vals_hbm.at[pl.ds(toff, CT)], vals_t, sem_t).wait()
    pltpu.async_copy(xs.at[src_t], rows_t, sem_t)
    pltpu.make_async_copy(xs.at[src_t], rows_t, sem_t).wait()
    for g in range(CT // 16):
        vgrp_t = vals_t[pl.ds(g * 16, 16)]
        for u in range(16):
            e = g * 16 + u
            vv = vgrp_t[u]
            for f in range(DH // 16):
                sl = pl.ds(f * 16, 16)
                rows_t[e, sl] = rows_t[e, sl] * vv
    pltpu.async_copy(rows_t, acc.at[dst_t], sem_t, add=True)

    wait_scatter(2)
    wait_scatter(3)
    pltpu.make_async_copy(rows_t, acc.at[dst_t], sem_t).wait()

    plsc.subcore_barrier()

    @pl.when(s < NS - 1)
    def _():
        pltpu.sync_copy(acc.at[pl.ds(s * ROWS_TILE, ROWS_TILE)],
                        out_hbm.at[pl.ds(s * ROWS_TILE, ROWS_TILE),
                                   pl.ds(c * DH, DH)])

    @pl.when(s == NS - 1)
    def _():
        pltpu.sync_copy(acc.at[pl.ds((NS - 1) * ROWS_TILE, ROWS_LAST)],
                        out_hbm.at[pl.ds((NS - 1) * ROWS_TILE, ROWS_LAST),
                                   pl.ds(c * DH, DH)])


@jax.jit
def _spmm(x, adj, vals):
    z = jnp.zeros((ROWS_TILE, DH), jnp.float32)

    mesh = plsc.VectorSubcoreMesh(core_axis_name="c", subcore_axis_name="s")
    out = pl.kernel(
        _body,
        out_type=jax.ShapeDtypeStruct((N_NODES, D), jnp.float32),
        mesh=mesh,
        compiler_params=pltpu.CompilerParams(use_tc_tiling_on_sc=False),
        scratch_types=[
            pltpu.VMEM((4, 2, CH), jnp.int32),
            pltpu.VMEM((4, CH), jnp.float32),
            pltpu.VMEM((4, CH, DH), jnp.float32),
            pltpu.VMEM((CT,), jnp.int32),
            pltpu.VMEM((CT,), jnp.int32),
            pltpu.VMEM((CT,), jnp.float32),
            pltpu.VMEM((CT, DH), jnp.float32),
            pltpu.VMEM_SHARED((N_PAD, DH), jnp.float32),
            pltpu.VMEM_SHARED((N_PAD, DH), jnp.float32),
            pltpu.SemaphoreType.DMA((4,)),
            pltpu.SemaphoreType.DMA((4,)),
            pltpu.SemaphoreType.DMA((4,)),
            pltpu.SemaphoreType.DMA,
        ],
    )(x, adj, vals, z)
    return out


def kernel(x, adj_indices, adj_values, idx):
    del idx
    return _spmm(x, adj_indices.astype(jnp.int32), adj_values)
